# baseline (speedup 1.0000x reference)
import math

import jax
import jax.numpy as jnp
import numpy as np
from jax import lax
from jax.experimental import pallas as pl

B = 16
L = 720
N = 862
NP = 896
D2 = 64
DM = 128
PN = 45
PLEN = 16
PRED = 96
EPS = 1e-5


def _dot(a, b, ca, cb):
    return lax.dot_general(a, b, (((ca,), (cb,)), ((), ())),
                           preferred_element_type=jnp.float32)


def _top3_softmax(s):
    m1 = jnp.max(s, axis=1, keepdims=True)
    s1 = jnp.where(s >= m1, jnp.float32(-1e30), s)
    m2 = jnp.max(s1, axis=1, keepdims=True)
    s2 = jnp.where(s1 >= m2, jnp.float32(-1e30), s1)
    m3 = jnp.max(s2, axis=1, keepdims=True)
    sm = jnp.where(s >= m3, s, jnp.float32(-1e9))
    e = jnp.exp(sm - m1)
    return e / jnp.sum(e, axis=1, keepdims=True)


def _body(x_ref, rw_ref, rb_ref, wep_ref, pe2_ref, psel_ref, cpw_ref,
          bt0rep_ref, wc_ref, wt1_ref, bt1_ref, wsp_ref, bsp_ref, wfc_ref,
          bfc_ref, wve_ref, bve_ref, wv0_ref, bv0_ref, wv1_ref, bv1_ref,
          wfc2_ref, bfc2_ref, w3a_ref, w3b_ref, bfc3_ref, out_ref):
    xb = x_ref[0]
    col = lax.broadcasted_iota(jnp.int32, (L, NP), 1)
    xb = jnp.where(col < N, xb, 0.0)
    mean = jnp.sum(xb, axis=0, keepdims=True) * (1.0 / L)
    sq = jnp.sum(xb * xb, axis=0, keepdims=True) * (1.0 / L)
    std = jnp.sqrt(sq - mean * mean + EPS)
    rw = rw_ref[0]
    rb = rb_ref[0]
    c = rw / (std * N)
    colv = lax.broadcasted_iota(jnp.int32, (1, NP), 1)
    c = jnp.where(colv < N, c, 0.0)
    mcol = jnp.sum(xb * c, axis=1, keepdims=True)
    off = jnp.sum(rb * jnp.where(colv < N, 1.0, 0.0)) * (1.0 / N) \
        - jnp.sum(mean * c)
    z = mcol + off
    zg = z * wep_ref[...]
    q = _dot(psel_ref[...], zg, 1, 0) + pe2_ref[...]
    s = _dot(q, q, 1, 1) * (1.0 / math.sqrt(D2))
    a = _top3_softmax(s)
    xn = (xb - mean) / std * rw + rb
    xn = jnp.where(col < N, xn, 0.0)
    ev = _dot(xn, wve_ref[...], 0, 0) + bve_ref[...]
    x3 = xn.reshape(PN, PLEN, NP)
    wc = wc_ref[...]
    cpw = cpw_ref[...]
    us = []
    for p in range(PN):
        us.append(_dot(wc, x3[p], 0, 0) + cpw[p][:, None])
    e = jnp.stack(us).reshape(PN, D2 * NP)
    g1 = jax.nn.gelu(_dot(a, e, 1, 0) + bt0rep_ref[...])
    av = _dot(a, g1, 1, 0).reshape(PN, D2, NP)
    wt1 = wt1_ref[...]
    bt1 = bt1_ref[...]
    u2 = []
    for p in range(PN):
        u2.append(jax.nn.gelu(_dot(wt1, av[p], 0, 0) + bt1[0][:, None]))
    vflat = jnp.concatenate(u2, axis=0)
    dec_time = _dot(vflat, wsp_ref[...], 0, 0) + bsp_ref[...]
    dt = _dot(dec_time, wfc_ref[...], 1, 0) + bfc_ref[...]
    rows = []
    for i in range(7):
        evc = ev[i * 128:(i + 1) * 128]
        sc = _dot(evc, ev, 1, 1) * (1.0 / math.sqrt(DM))
        colm = lax.broadcasted_iota(jnp.int32, (128, NP), 1)
        sc = jnp.where(colm < N, sc, jnp.float32(-1e9))
        rows.append(_top3_softmax(sc))
    avar = jnp.concatenate(rows, axis=0)
    h = ev
    for wv, bv in ((wv0_ref, bv0_ref), (wv1_ref, bv1_ref)):
        h = jax.nn.gelu(_dot(_dot(avar, h, 1, 0), wv[...], 1, 0) + bv[...])
    dv = _dot(h, wfc2_ref[...], 1, 0) + bfc2_ref[...]
    dec = _dot(dt, w3a_ref[...], 1, 0) + _dot(dv, w3b_ref[...], 1, 0) \
        + bfc3_ref[...]
    dec = (dec - rb[0][:, None]) / (rw[0][:, None] + EPS * EPS) \
        * std[0][:, None] + mean[0][:, None]
    out_ref[...] = jnp.transpose(dec)[None]


def _pe2_const():
    pos = np.arange(PN)[:, None].astype(np.float32)
    div = np.exp(np.arange(0, D2, 2).astype(np.float32) * -(np.log(10000.0) / D2))
    pe = np.zeros((PN, D2), dtype=np.float32)
    pe[:, 0::2] = np.sin(pos * div)
    pe[:, 1::2] = np.cos(pos * div)
    return pe


@jax.jit
def kernel(x_enc, rev_w, rev_b, W_ep, b_ep, Wt, bt, W_sp, b_sp, W_ve, b_ve,
           Wv, bv, W_fc, b_fc, W_fc2, b_fc2, W_fc3, b_fc3):
    f32 = jnp.float32
    rw = jnp.pad(rev_w, (0, NP - N)).reshape(1, 1, NP)
    rb = jnp.pad(rev_b, (0, NP - N)).reshape(1, 1, NP)
    pe2 = jnp.asarray(_pe2_const()) + b_ep[None, :]
    wep_tiled = jnp.tile(W_ep, (PN, 1))
    psel = jnp.asarray(np.kron(np.eye(PN, dtype=np.float32),
                               np.ones((1, PLEN), np.float32)))
    wc = W_ep @ Wt[0]
    cpw = pe2 @ Wt[0]
    bt0rep = jnp.tile(bt[0][:, None], (1, NP)).reshape(1, D2 * NP)
    w3a, w3b = W_fc3[:PRED], W_fc3[PRED:]

    def w(shape):
        return pl.BlockSpec(shape, lambda b: tuple(0 for _ in shape))

    out = pl.pallas_call(
        _body,
        grid=(B,),
        in_specs=[
            pl.BlockSpec((1, L, NP), lambda b: (b, 0, 0)),
            w((1, 1, NP)), w((1, 1, NP)),
            w((L, D2)), w((PN, D2)), w((PN, L)), w((PN, D2)),
            w((1, D2 * NP)), w((PLEN, D2)), w((D2, D2)), w((1, D2)),
            w((PN * D2, DM)), w((1, DM)), w((DM, PRED)), w((1, PRED)),
            w((L, DM)), w((1, DM)),
            w((DM, DM)), w((1, DM)), w((DM, DM)), w((1, DM)),
            w((DM, PRED)), w((1, PRED)),
            w((PRED, PRED)), w((PRED, PRED)), w((1, PRED)),
        ],
        out_specs=[pl.BlockSpec((1, PRED, NP), lambda b: (b, 0, 0))],
        out_shape=[jax.ShapeDtypeStruct((B, PRED, NP), f32)],
    )(x_enc, rw, rb, wep_tiled, pe2, psel, cpw, bt0rep,
      wc, Wt[1], bt[1].reshape(1, D2),
      W_sp, b_sp.reshape(1, DM), W_fc, b_fc.reshape(1, PRED),
      W_ve, b_ve.reshape(1, DM),
      Wv[0], bv[0].reshape(1, DM), Wv[1], bv[1].reshape(1, DM),
      W_fc2, b_fc2.reshape(1, PRED), w3a, w3b, b_fc3.reshape(1, PRED))[0]

    return out[:, :, :N]


# speedup vs baseline: 1.0171x; 1.0171x over previous
import math

import jax
import jax.numpy as jnp
import numpy as np
from jax import lax
from jax.experimental import pallas as pl

B = 16
L = 720
N = 862
NP = 896
D2 = 64
DM = 128
PN = 45
PLEN = 16
PRED = 96
EPS = 1e-5


def _dot(a, b, ca, cb):
    return lax.dot_general(a, b, (((ca,), (cb,)), ((), ())),
                           preferred_element_type=jnp.float32)


def _top3_softmax(s):
    m1 = jnp.max(s, axis=1, keepdims=True)
    s1 = jnp.where(s >= m1, jnp.float32(-1e30), s)
    m2 = jnp.max(s1, axis=1, keepdims=True)
    s2 = jnp.where(s1 >= m2, jnp.float32(-1e30), s1)
    m3 = jnp.max(s2, axis=1, keepdims=True)
    sm = jnp.where(s >= m3, s, jnp.float32(-1e9))
    e = jnp.exp(sm - m1)
    return e / jnp.sum(e, axis=1, keepdims=True)


def _body(x_ref, rw_ref, rb_ref, wep_ref, pe2_ref, psel_ref, wt0_ref, bt0_ref,
          p16_ref, wc_ref, wt1_ref, bt1_ref, wsp_ref, bsp_ref, wfc_ref,
          bfc_ref, wve_ref, bve_ref, wv0_ref, bv0_ref, wv1_ref, bv1_ref,
          wfc2_ref, bfc2_ref, w3a_ref, w3b_ref, bfc3_ref, out_ref):
    xb = x_ref[0]
    col = lax.broadcasted_iota(jnp.int32, (L, NP), 1)
    xb = jnp.where(col < N, xb, 0.0)
    mean = jnp.sum(xb, axis=0, keepdims=True) * (1.0 / L)
    sq = jnp.sum(xb * xb, axis=0, keepdims=True) * (1.0 / L)
    std = jnp.sqrt(sq - mean * mean + EPS)
    rw = rw_ref[0]
    rb = rb_ref[0]
    c = rw / (std * N)
    colv = lax.broadcasted_iota(jnp.int32, (1, NP), 1)
    c = jnp.where(colv < N, c, 0.0)
    mcol = jnp.sum(xb * c, axis=1, keepdims=True)
    off = jnp.sum(rb * jnp.where(colv < N, 1.0, 0.0)) * (1.0 / N) \
        - jnp.sum(mean * c)
    z = mcol + off
    zg = z * wep_ref[...]
    q = _dot(psel_ref[...], zg, 1, 0) + pe2_ref[...]
    s = _dot(q, q, 1, 1) * (1.0 / math.sqrt(D2))
    a = _top3_softmax(s)
    pb1 = _dot(_dot(a, pe2_ref[...], 1, 0), wt0_ref[...], 1, 0) + bt0_ref[...]
    p16 = p16_ref[...]
    full = _dot(_dot(p16, a, 1, 0), p16, 1, 1)
    ri = lax.broadcasted_iota(jnp.int32, (L, L), 0)
    ci = lax.broadcasted_iota(jnp.int32, (L, L), 1)
    a16 = jnp.where((ri % PLEN) == (ci % PLEN), full, 0.0)
    xn = (xb - mean) / std * rw + rb
    xn = jnp.where(col < N, xn, 0.0)
    ev = _dot(xn, wve_ref[...], 0, 0) + bve_ref[...]
    xmix = _dot(a16, xn, 1, 0)
    xm3 = xmix.reshape(PN, PLEN, NP)
    wc = wc_ref[...]
    us = []
    for p in range(PN):
        us.append(jax.nn.gelu(_dot(wc, xm3[p], 0, 0) + pb1[p][:, None]))
    v = jnp.stack(us)
    av = _dot(a, v.reshape(PN, D2 * NP), 1, 0).reshape(PN, D2, NP)
    wt1 = wt1_ref[...]
    bt1 = bt1_ref[...]
    u2 = []
    for p in range(PN):
        u2.append(jax.nn.gelu(_dot(wt1, av[p], 0, 0) + bt1[0][:, None]))
    vflat = jnp.concatenate(u2, axis=0)
    dec_time = _dot(vflat, wsp_ref[...], 0, 0) + bsp_ref[...]
    dt = _dot(dec_time, wfc_ref[...], 1, 0) + bfc_ref[...]
    rows = []
    for i in range(7):
        evc = ev[i * 128:(i + 1) * 128]
        sc = _dot(evc, ev, 1, 1) * (1.0 / math.sqrt(DM))
        colm = lax.broadcasted_iota(jnp.int32, (128, NP), 1)
        sc = jnp.where(colm < N, sc, jnp.float32(-1e9))
        rows.append(_top3_softmax(sc))
    avar = jnp.concatenate(rows, axis=0)
    h = ev
    for wv, bv in ((wv0_ref, bv0_ref), (wv1_ref, bv1_ref)):
        h = jax.nn.gelu(_dot(_dot(avar, h, 1, 0), wv[...], 1, 0) + bv[...])
    dv = _dot(h, wfc2_ref[...], 1, 0) + bfc2_ref[...]
    dec = _dot(dt, w3a_ref[...], 1, 0) + _dot(dv, w3b_ref[...], 1, 0) \
        + bfc3_ref[...]
    dec = (dec - rb[0][:, None]) / (rw[0][:, None] + EPS * EPS) \
        * std[0][:, None] + mean[0][:, None]
    out_ref[...] = jnp.transpose(dec)[None]


def _pe2_const():
    pos = np.arange(PN)[:, None].astype(np.float32)
    div = np.exp(np.arange(0, D2, 2).astype(np.float32) * -(np.log(10000.0) / D2))
    pe = np.zeros((PN, D2), dtype=np.float32)
    pe[:, 0::2] = np.sin(pos * div)
    pe[:, 1::2] = np.cos(pos * div)
    return pe


@jax.jit
def kernel(x_enc, rev_w, rev_b, W_ep, b_ep, Wt, bt, W_sp, b_sp, W_ve, b_ve,
           Wv, bv, W_fc, b_fc, W_fc2, b_fc2, W_fc3, b_fc3):
    f32 = jnp.float32
    rw = jnp.pad(rev_w, (0, NP - N)).reshape(1, 1, NP)
    rb = jnp.pad(rev_b, (0, NP - N)).reshape(1, 1, NP)
    pe2 = jnp.asarray(_pe2_const()) + b_ep[None, :]
    wep_tiled = jnp.tile(W_ep, (PN, 1))
    psel = jnp.asarray(np.kron(np.eye(PN, dtype=np.float32),
                               np.ones((1, PLEN), np.float32)))
    p16 = jnp.asarray(np.kron(np.eye(PN, dtype=np.float32),
                              np.ones((PLEN, 1), np.float32)))
    wc = W_ep @ Wt[0]
    w3a, w3b = W_fc3[:PRED], W_fc3[PRED:]

    def w(shape):
        return pl.BlockSpec(shape, lambda b: tuple(0 for _ in shape))

    out = pl.pallas_call(
        _body,
        grid=(B,),
        in_specs=[
            pl.BlockSpec((1, L, NP), lambda b: (b, 0, 0)),
            w((1, 1, NP)), w((1, 1, NP)),
            w((L, D2)), w((PN, D2)), w((PN, L)), w((D2, D2)), w((1, D2)),
            w((L, PN)), w((PLEN, D2)), w((D2, D2)), w((1, D2)),
            w((PN * D2, DM)), w((1, DM)), w((DM, PRED)), w((1, PRED)),
            w((L, DM)), w((1, DM)),
            w((DM, DM)), w((1, DM)), w((DM, DM)), w((1, DM)),
            w((DM, PRED)), w((1, PRED)),
            w((PRED, PRED)), w((PRED, PRED)), w((1, PRED)),
        ],
        out_specs=[pl.BlockSpec((1, PRED, NP), lambda b: (b, 0, 0))],
        out_shape=[jax.ShapeDtypeStruct((B, PRED, N), f32)],
    )(x_enc, rw, rb, wep_tiled, pe2, psel, Wt[0], bt[0].reshape(1, D2),
      p16, wc, Wt[1], bt[1].reshape(1, D2),
      W_sp, b_sp.reshape(1, DM), W_fc, b_fc.reshape(1, PRED),
      W_ve, b_ve.reshape(1, DM),
      Wv[0], bv[0].reshape(1, DM), Wv[1], bv[1].reshape(1, DM),
      W_fc2, b_fc2.reshape(1, PRED), w3a, w3b, b_fc3.reshape(1, PRED))[0]

    return out


# speedup vs baseline: 1.0171x; 1.0001x over previous
import math

import jax
import jax.numpy as jnp
import numpy as np
from jax import lax
from jax.experimental import pallas as pl
from jax.experimental.pallas import tpu as pltpu

B = 16
L = 720
N = 862
NP = 896
D2 = 64
DM = 128
PN = 45
PLEN = 16
PRED = 96
EPS = 1e-5


def _dot(a, b, ca, cb):
    return lax.dot_general(a, b, (((ca,), (cb,)), ((), ())),
                           preferred_element_type=jnp.float32)


def _top3_softmax(s):
    m1 = jnp.max(s, axis=1, keepdims=True)
    s1 = jnp.where(s >= m1, jnp.float32(-1e30), s)
    m2 = jnp.max(s1, axis=1, keepdims=True)
    s2 = jnp.where(s1 >= m2, jnp.float32(-1e30), s1)
    m3 = jnp.max(s2, axis=1, keepdims=True)
    sm = jnp.where(s >= m3, s, jnp.float32(-1e9))
    e = jnp.exp(sm - m1)
    return e / jnp.sum(e, axis=1, keepdims=True)


def _body(x_ref, rw_ref, rb_ref, wep_ref, pe2_ref, psel_ref, wt0_ref, bt0_ref,
          p16_ref, wc_ref, wt1_ref, bt1_ref, wsp_ref, bsp_ref, wfc_ref,
          bfc_ref, wve_ref, bve_ref, wv0_ref, bv0_ref, wv1_ref, bv1_ref,
          wfc2_ref, bfc2_ref, w3a_ref, w3b_ref, bfc3_ref, out_ref):
    xb = x_ref[0]
    col = lax.broadcasted_iota(jnp.int32, (L, NP), 1)
    xb = jnp.where(col < N, xb, 0.0)
    mean = jnp.sum(xb, axis=0, keepdims=True) * (1.0 / L)
    sq = jnp.sum(xb * xb, axis=0, keepdims=True) * (1.0 / L)
    std = jnp.sqrt(sq - mean * mean + EPS)
    rw = rw_ref[0]
    rb = rb_ref[0]
    c = rw / (std * N)
    colv = lax.broadcasted_iota(jnp.int32, (1, NP), 1)
    c = jnp.where(colv < N, c, 0.0)
    mcol = jnp.sum(xb * c, axis=1, keepdims=True)
    off = jnp.sum(rb * jnp.where(colv < N, 1.0, 0.0)) * (1.0 / N) \
        - jnp.sum(mean * c)
    z = mcol + off
    zg = z * wep_ref[...]
    q = _dot(psel_ref[...], zg, 1, 0) + pe2_ref[...]
    s = _dot(q, q, 1, 1) * (1.0 / math.sqrt(D2))
    a = _top3_softmax(s)
    pb1 = _dot(_dot(a, pe2_ref[...], 1, 0), wt0_ref[...], 1, 0) + bt0_ref[...]
    p16 = p16_ref[...]
    full = _dot(_dot(p16, a, 1, 0), p16, 1, 1)
    ri = lax.broadcasted_iota(jnp.int32, (L, L), 0)
    ci = lax.broadcasted_iota(jnp.int32, (L, L), 1)
    a16 = jnp.where((ri % PLEN) == (ci % PLEN), full, 0.0)
    xn = (xb - mean) / std * rw + rb
    xn = jnp.where(col < N, xn, 0.0)
    ev = _dot(xn, wve_ref[...], 0, 0) + bve_ref[...]
    xmix = _dot(a16, xn, 1, 0)
    xm3 = xmix.reshape(PN, PLEN, NP)
    wc = wc_ref[...]
    us = []
    for p in range(PN):
        us.append(jax.nn.gelu(_dot(wc, xm3[p], 0, 0) + pb1[p][:, None]))
    v = jnp.stack(us)
    av = _dot(a, v.reshape(PN, D2 * NP), 1, 0).reshape(PN, D2, NP)
    wt1 = wt1_ref[...]
    bt1 = bt1_ref[...]
    u2 = []
    for p in range(PN):
        u2.append(jax.nn.gelu(_dot(wt1, av[p], 0, 0) + bt1[0][:, None]))
    vflat = jnp.concatenate(u2, axis=0)
    dec_time = _dot(vflat, wsp_ref[...], 0, 0) + bsp_ref[...]
    dt = _dot(dec_time, wfc_ref[...], 1, 0) + bfc_ref[...]
    rows = []
    for i in range(7):
        evc = ev[i * 128:(i + 1) * 128]
        sc = _dot(evc, ev, 1, 1) * (1.0 / math.sqrt(DM))
        colm = lax.broadcasted_iota(jnp.int32, (128, NP), 1)
        sc = jnp.where(colm < N, sc, jnp.float32(-1e9))
        rows.append(_top3_softmax(sc))
    avar = jnp.concatenate(rows, axis=0)
    h = ev
    for wv, bv in ((wv0_ref, bv0_ref), (wv1_ref, bv1_ref)):
        h = jax.nn.gelu(_dot(_dot(avar, h, 1, 0), wv[...], 1, 0) + bv[...])
    dv = _dot(h, wfc2_ref[...], 1, 0) + bfc2_ref[...]
    dec = _dot(dt, w3a_ref[...], 1, 0) + _dot(dv, w3b_ref[...], 1, 0) \
        + bfc3_ref[...]
    dec = (dec - rb[0][:, None]) / (rw[0][:, None] + EPS * EPS) \
        * std[0][:, None] + mean[0][:, None]
    out_ref[...] = jnp.transpose(dec)[None]


def _pe2_const():
    pos = np.arange(PN)[:, None].astype(np.float32)
    div = np.exp(np.arange(0, D2, 2).astype(np.float32) * -(np.log(10000.0) / D2))
    pe = np.zeros((PN, D2), dtype=np.float32)
    pe[:, 0::2] = np.sin(pos * div)
    pe[:, 1::2] = np.cos(pos * div)
    return pe


@jax.jit
def kernel(x_enc, rev_w, rev_b, W_ep, b_ep, Wt, bt, W_sp, b_sp, W_ve, b_ve,
           Wv, bv, W_fc, b_fc, W_fc2, b_fc2, W_fc3, b_fc3):
    f32 = jnp.float32
    rw = jnp.pad(rev_w, (0, NP - N)).reshape(1, 1, NP)
    rb = jnp.pad(rev_b, (0, NP - N)).reshape(1, 1, NP)
    pe2 = jnp.asarray(_pe2_const()) + b_ep[None, :]
    wep_tiled = jnp.tile(W_ep, (PN, 1))
    psel = jnp.asarray(np.kron(np.eye(PN, dtype=np.float32),
                               np.ones((1, PLEN), np.float32)))
    p16 = jnp.asarray(np.kron(np.eye(PN, dtype=np.float32),
                              np.ones((PLEN, 1), np.float32)))
    wc = W_ep @ Wt[0]
    w3a, w3b = W_fc3[:PRED], W_fc3[PRED:]

    def w(shape):
        return pl.BlockSpec(shape, lambda b: tuple(0 for _ in shape))

    out = pl.pallas_call(
        _body,
        grid=(B,),
        in_specs=[
            pl.BlockSpec((1, L, NP), lambda b: (b, 0, 0)),
            w((1, 1, NP)), w((1, 1, NP)),
            w((L, D2)), w((PN, D2)), w((PN, L)), w((D2, D2)), w((1, D2)),
            w((L, PN)), w((PLEN, D2)), w((D2, D2)), w((1, D2)),
            w((PN * D2, DM)), w((1, DM)), w((DM, PRED)), w((1, PRED)),
            w((L, DM)), w((1, DM)),
            w((DM, DM)), w((1, DM)), w((DM, DM)), w((1, DM)),
            w((DM, PRED)), w((1, PRED)),
            w((PRED, PRED)), w((PRED, PRED)), w((1, PRED)),
        ],
        out_specs=[pl.BlockSpec((1, PRED, NP), lambda b: (b, 0, 0))],
        out_shape=[jax.ShapeDtypeStruct((B, PRED, N), f32)],
        compiler_params=pltpu.CompilerParams(
            dimension_semantics=("parallel",)),
    )(x_enc, rw, rb, wep_tiled, pe2, psel, Wt[0], bt[0].reshape(1, D2),
      p16, wc, Wt[1], bt[1].reshape(1, D2),
      W_sp, b_sp.reshape(1, DM), W_fc, b_fc.reshape(1, PRED),
      W_ve, b_ve.reshape(1, DM),
      Wv[0], bv[0].reshape(1, DM), Wv[1], bv[1].reshape(1, DM),
      W_fc2, b_fc2.reshape(1, PRED), w3a, w3b, b_fc3.reshape(1, PRED))[0]

    return out
